# baseline (speedup 1.0000x reference)
import functools

import jax
import jax.numpy as jnp
from jax import lax
from jax.experimental import pallas as pl
from jax.experimental.pallas import tpu as pltpu

_B, _L, _D = 4096, 200, 128
_BB = 128
_K = 4


def _tc_body(tab_ref, out_ref, buf_a, buf_b, sem):
    buf_a[...] = jnp.broadcast_to(tab_ref[...][None, :, :], (_BB, _L, _D))
    buf_b[...] = jnp.broadcast_to(tab_ref[...][None, :, :], (_BB, _L, _D))
    bufs = [buf_a, buf_b]

    def chunk(j, c):
        row = j * (_K * _BB)
        for t in range(_K):
            pltpu.make_async_copy(
                bufs[t % 2], out_ref.at[pl.ds(row + t * _BB, _BB)], sem
            ).start()
        for t in range(_K):
            pltpu.make_async_copy(
                bufs[t % 2], out_ref.at[pl.ds(row + t * _BB, _BB)], sem
            ).wait()
        return c

    lax.fori_loop(0, _B // (_K * _BB), chunk, 0)


def kernel(sequence, table):
    return pl.pallas_call(
        _tc_body,
        out_specs=pl.BlockSpec(memory_space=pl.ANY),
        out_shape=jax.ShapeDtypeStruct((_B, _L, _D), table.dtype),
        scratch_shapes=[
            pltpu.VMEM((_BB, _L, _D), jnp.float32),
            pltpu.VMEM((_BB, _L, _D), jnp.float32),
            pltpu.SemaphoreType.DMA,
        ],
    )(table[:_L])
